# baseline (speedup 1.0000x reference)
import functools

import jax
import jax.numpy as jnp
from jax import lax
from jax.experimental import pallas as pl
from jax.experimental.pallas import tpu as pltpu
from jax.experimental.pallas import tpu_sc as plsc

N_NODES = 10000
N_EDGES = 320000
D = 128
N_ETYPES = 8

NUM_WORKERS = 32
EPW = N_EDGES // NUM_WORKERS
CHUNK = 80
NUM_CHUNKS = EPW // CHUNK
GROUPS = CHUNK // 16
_EXP_NO_DMA = False

TC_ROWS = 2000


def _tc_premul_body(h_ref, rel_ref, out_ref):
    r = pl.program_id(0)
    out_ref[...] = h_ref[...] * rel_ref[pl.ds(r, 1), :]


def _build_ht(h, rel_weight):
    nb = N_NODES // TC_ROWS
    return pl.pallas_call(
        _tc_premul_body,
        out_shape=jax.ShapeDtypeStruct((N_ETYPES * N_NODES, D), jnp.float32),
        grid=(N_ETYPES, nb),
        in_specs=[
            pl.BlockSpec((TC_ROWS, D), lambda r, b: (b, 0)),
            pl.BlockSpec((N_ETYPES, D), lambda r, b: (0, 0)),
        ],
        out_specs=pl.BlockSpec((TC_ROWS, D), lambda r, b, _nb=nb: (r * _nb + b, 0)),
    )(h, rel_weight)


W32 = D // 2


def _sc_body(ht_hbm, h_hbm, u_hbm, v_hbm, et_hbm, out_hbm,
             idx_u, idx_v, et_v, rows_u, rows_v, out_v,
             sem_u, sem_v):
    cid = lax.axis_index("c")
    sid = lax.axis_index("s")
    wid = sid * 2 + cid
    wbase = wid * EPW

    pltpu.sync_copy(u_hbm.at[pl.ds(wbase, EPW)], idx_u)
    pltpu.sync_copy(v_hbm.at[pl.ds(wbase, EPW)], idx_v)
    pltpu.sync_copy(et_hbm.at[pl.ds(wbase, EPW)], et_v)

    def idx_body(g, carry):
        sl = pl.ds(g * 16, 16)
        idx_u[sl] = et_v[sl] * N_NODES + idx_u[sl]
        return carry

    lax.fori_loop(0, EPW // 16, idx_body, 0)

    def issue(i, b):
        if _EXP_NO_DMA:
            return
        pltpu.async_copy(ht_hbm.at[idx_u.at[pl.ds(i * CHUNK, CHUNK)]],
                         rows_u.at[b], sem_u.at[b])
        pltpu.async_copy(h_hbm.at[idx_v.at[pl.ds(i * CHUNK, CHUNK)]],
                         rows_v.at[b], sem_v.at[b])

    def wait(b):
        if _EXP_NO_DMA:
            return
        dummy = h_hbm.at[pl.ds(0, CHUNK)]
        pltpu.make_async_copy(dummy, rows_u.at[b], sem_u.at[b]).wait()
        pltpu.make_async_copy(dummy, rows_v.at[b], sem_v.at[b]).wait()

    lane = lax.iota(jnp.int32, 16)
    rot = [(lane + t) & 15 for t in range(16)]

    def compute(i, b):
        @plsc.parallel_loop(0, GROUPS)
        def group_body(g):
            e16 = g * 16 + lane
            accs = [jnp.zeros((16,), jnp.float32) for _ in range(4)]
            n = 0
            for jb in range(D // 16):
                for t in range(16):
                    dv = rot[t] + (jb * 16)
                    xu = plsc.load_gather(rows_u.at[b], [e16, dv])
                    xv = plsc.load_gather(rows_v.at[b], [e16, dv])
                    accs[n % 4] = accs[n % 4] + xu * xv
                    n += 1
            score = (accs[0] + accs[1]) + (accs[2] + accs[3])
            out_v[pl.ds(i * CHUNK + g * 16, 16)] = score

    issue(0, 0)

    def pair_body(p, carry):
        i = p * 2
        wait(0)
        issue(i + 1, 1)
        compute(i, 0)
        wait(1)

        @pl.when(i + 2 < NUM_CHUNKS)
        def _():
            issue(i + 2, 0)

        compute(i + 1, 1)
        return carry

    lax.fori_loop(0, NUM_CHUNKS // 2, pair_body, 0)
    wait(0)
    compute(NUM_CHUNKS - 1, 0)

    @plsc.parallel_loop(0, EPW // 16)
    def sig_body(g):
        x = out_v[pl.ds(g * 16, 16)]
        out_v[pl.ds(g * 16, 16)] = 1.0 / (1.0 + jnp.exp(-x))
    pltpu.sync_copy(out_v, out_hbm.at[pl.ds(wbase, EPW)])


@jax.jit
def _dist_mul_sc(h, u, v, etype, rel_weight):
    ht = _build_ht(h, rel_weight)
    hb = h
    mesh = plsc.VectorSubcoreMesh(core_axis_name="c", subcore_axis_name="s")
    return pl.kernel(
        _sc_body,
        out_type=jax.ShapeDtypeStruct((N_EDGES,), jnp.float32),
        mesh=mesh,
        scratch_types=[
            pltpu.VMEM((EPW,), jnp.int32),
            pltpu.VMEM((EPW,), jnp.int32),
            pltpu.VMEM((EPW,), jnp.int32),
            pltpu.VMEM((2, CHUNK, D), jnp.float32),
            pltpu.VMEM((2, CHUNK, D), jnp.float32),
            pltpu.VMEM((EPW,), jnp.float32),
            pltpu.SemaphoreType.DMA((2,)),
            pltpu.SemaphoreType.DMA((2,)),
        ],
        compiler_params=pltpu.CompilerParams(needs_layout_passes=False),
    )(ht, hb, u, v, etype)


def kernel(h, u, v, etype, rel_weight):
    u = u.astype(jnp.int32)
    v = v.astype(jnp.int32)
    etype = etype.astype(jnp.int32)
    return _dist_mul_sc(h, u, v, etype, rel_weight)


# speedup vs baseline: 2.0680x; 2.0680x over previous
import functools

import jax
import jax.numpy as jnp
from jax import lax
from jax.experimental import pallas as pl
from jax.experimental.pallas import tpu as pltpu
from jax.experimental.pallas import tpu_sc as plsc

N_NODES = 10000
N_EDGES = 320000
D = 128
N_ETYPES = 8

NUM_WORKERS = 32
EPW = N_EDGES // NUM_WORKERS
CHUNK = 80
NUM_CHUNKS = EPW // CHUNK
GROUPS = CHUNK // 16
_EXP_NO_DMA = False

TC_ROWS = 2000


def _tc_premul_body(h_ref, rel_ref, out_ref):
    r = pl.program_id(0)
    out_ref[...] = h_ref[...] * rel_ref[pl.ds(r, 1), :]


def _build_ht(h, rel_weight):
    nb = N_NODES // TC_ROWS
    return pl.pallas_call(
        _tc_premul_body,
        out_shape=jax.ShapeDtypeStruct((N_ETYPES * N_NODES, D), jnp.float32),
        grid=(N_ETYPES, nb),
        in_specs=[
            pl.BlockSpec((TC_ROWS, D), lambda r, b: (b, 0)),
            pl.BlockSpec((N_ETYPES, D), lambda r, b: (0, 0)),
        ],
        out_specs=pl.BlockSpec((TC_ROWS, D), lambda r, b, _nb=nb: (r * _nb + b, 0)),
    )(h, rel_weight)


W32 = D // 2


def _sc_body(ht_hbm, h_hbm, u_hbm, v_hbm, et_hbm, out_hbm,
             idx_u, idx_v, et_v, rows_u, rows_v, out_v,
             sem_u, sem_v):
    cid = lax.axis_index("c")
    sid = lax.axis_index("s")
    wid = sid * 2 + cid
    wbase = wid * EPW

    pltpu.sync_copy(u_hbm.at[pl.ds(wbase, EPW)], idx_u)
    pltpu.sync_copy(v_hbm.at[pl.ds(wbase, EPW)], idx_v)
    pltpu.sync_copy(et_hbm.at[pl.ds(wbase, EPW)], et_v)

    def idx_body(g, carry):
        sl = pl.ds(g * 16, 16)
        idx_u[sl] = et_v[sl] * N_NODES + idx_u[sl]
        return carry

    lax.fori_loop(0, EPW // 16, idx_body, 0)

    def issue(i, b):
        if _EXP_NO_DMA:
            return
        pltpu.async_copy(ht_hbm.at[idx_u.at[pl.ds(i * CHUNK, CHUNK)]],
                         rows_u.at[b], sem_u.at[b])
        pltpu.async_copy(h_hbm.at[idx_v.at[pl.ds(i * CHUNK, CHUNK)]],
                         rows_v.at[b], sem_v.at[b])

    def wait(b):
        if _EXP_NO_DMA:
            return
        dummy = h_hbm.at[pl.ds(0, CHUNK)]
        pltpu.make_async_copy(dummy, rows_u.at[b], sem_u.at[b]).wait()
        pltpu.make_async_copy(dummy, rows_v.at[b], sem_v.at[b]).wait()

    zeros16 = jnp.zeros((16,), jnp.float32)
    zeros16i = jnp.zeros((16,), jnp.int32)

    def compute(i, b):
        @plsc.parallel_loop(0, GROUPS)
        def group_body(g):
            base = i * CHUNK + g * 16
            out_v[pl.ds(base, 16)] = zeros16
            gb = zeros16i + base
            for k in range(16):
                e = g * 16 + k
                acc = None
                for j in range(D // 16):
                    sl = pl.ds(j * 16, 16)
                    prod = rows_u[b, e, sl] * rows_v[b, e, sl]
                    acc = prod if acc is None else acc + prod
                plsc.addupdate_scatter(out_v, [gb + k], acc)

    issue(0, 0)

    def pair_body(p, carry):
        i = p * 2
        wait(0)
        issue(i + 1, 1)
        compute(i, 0)
        wait(1)

        @pl.when(i + 2 < NUM_CHUNKS)
        def _():
            issue(i + 2, 0)

        compute(i + 1, 1)
        return carry

    lax.fori_loop(0, NUM_CHUNKS // 2, pair_body, 0)
    wait(0)
    compute(NUM_CHUNKS - 1, 0)

    @plsc.parallel_loop(0, EPW // 16)
    def sig_body(g):
        x = out_v[pl.ds(g * 16, 16)]
        out_v[pl.ds(g * 16, 16)] = 1.0 / (1.0 + jnp.exp(-x))
    pltpu.sync_copy(out_v, out_hbm.at[pl.ds(wbase, EPW)])


@jax.jit
def _dist_mul_sc(h, u, v, etype, rel_weight):
    ht = _build_ht(h, rel_weight)
    hb = h
    mesh = plsc.VectorSubcoreMesh(core_axis_name="c", subcore_axis_name="s")
    return pl.kernel(
        _sc_body,
        out_type=jax.ShapeDtypeStruct((N_EDGES,), jnp.float32),
        mesh=mesh,
        scratch_types=[
            pltpu.VMEM((EPW,), jnp.int32),
            pltpu.VMEM((EPW,), jnp.int32),
            pltpu.VMEM((EPW,), jnp.int32),
            pltpu.VMEM((2, CHUNK, D), jnp.float32),
            pltpu.VMEM((2, CHUNK, D), jnp.float32),
            pltpu.VMEM((EPW,), jnp.float32),
            pltpu.SemaphoreType.DMA((2,)),
            pltpu.SemaphoreType.DMA((2,)),
        ],
        compiler_params=pltpu.CompilerParams(needs_layout_passes=False),
    )(ht, hb, u, v, etype)


def kernel(h, u, v, etype, rel_weight):
    u = u.astype(jnp.int32)
    v = v.astype(jnp.int32)
    etype = etype.astype(jnp.int32)
    return _dist_mul_sc(h, u, v, etype, rel_weight)
